# speedup vs baseline: 7.1471x; 7.1471x over previous
import jax
import jax.numpy as jnp
from jax.experimental import pallas as pl
from jax.experimental.pallas import tpu as pltpu

D = 512
DH = 128
G = 512
BLK = 512


def _gating_body(idx_ref, x_ref, w1_ref, b1_ref, w2_ref, b2_ref,
                 scaled_ref, seg_ref, cnt_ref):
    x = x_ref[...]
    h = jax.lax.dot_general(x, w1_ref[...], (((1,), (0,)), ((), ())),
                            preferred_element_type=jnp.float32)
    h = jnp.maximum(h + b1_ref[...], 0.0)
    fc = jax.lax.dot_general(h, w2_ref[...], (((1,), (0,)), ((), ())),
                             preferred_element_type=jnp.float32)
    fc = jnp.tanh(fc + b2_ref[...])
    scaled = (fc + 1.0) * x
    scaled_ref[...] = scaled

    idx = idx_ref[0, 0, :]
    gids = jax.lax.broadcasted_iota(jnp.int32, (BLK, G), 1)
    onehot = (idx[:, None] == gids).astype(jnp.float32)
    seg_c = jax.lax.dot_general(onehot, scaled, (((0,), (0,)), ((), ())),
                                preferred_element_type=jnp.float32)
    cnt_c = jax.lax.dot_general(onehot, jnp.ones((BLK, 1), jnp.float32),
                                (((0,), (0,)), ((), ())),
                                preferred_element_type=jnp.float32)

    @pl.when(pl.program_id(0) == 0)
    def _init():
        seg_ref[...] = jnp.zeros_like(seg_ref)
        cnt_ref[...] = jnp.zeros_like(cnt_ref)

    seg_ref[...] += seg_c
    cnt_ref[...] += cnt_c


def _context_body(seg_ref, cnt_ref, wm_ref, gc_ref):
    mean = seg_ref[...] / cnt_ref[...]
    gc = jax.lax.dot_general(mean, wm_ref[...], (((1,), (1,)), ((), ())),
                             preferred_element_type=jnp.float32)
    gc_ref[...] = jnp.tanh(gc)


def _output_body(idx_ref, scaled_ref, gc_ref, out_ref):
    scaled = scaled_ref[...]
    idx = idx_ref[0, 0, :]
    gids = jax.lax.broadcasted_iota(jnp.int32, (BLK, G), 1)
    onehot = (idx[:, None] == gids).astype(jnp.float32)
    dots = jax.lax.dot_general(scaled, gc_ref[...], (((1,), (1,)), ((), ())),
                               preferred_element_type=jnp.float32)
    s = jnp.sum(dots * onehot, axis=1)
    coef = 1.0 / (1.0 + jnp.exp(-s))
    weighted = coef[:, None] * scaled
    out_c = jax.lax.dot_general(onehot, weighted, (((0,), (0,)), ((), ())),
                                preferred_element_type=jnp.float32)

    @pl.when(pl.program_id(0) == 0)
    def _init():
        out_ref[...] = jnp.zeros_like(out_ref)

    out_ref[...] += out_c


def kernel(node_features, graph_idx, num_graphs, W_g1, b_g1, W_g2, b_g2, W_mean):
    n = node_features.shape[0]
    nblk = pl.cdiv(n, BLK)
    npad = nblk * BLK
    x = jnp.pad(node_features.astype(jnp.float32), ((0, npad - n), (0, 0)))
    idx = jnp.pad(graph_idx.astype(jnp.int32), (0, npad - n), constant_values=G)
    idx3 = idx.reshape(nblk, 1, BLK)
    w1t = W_g1.T
    w2t = W_g2.T
    b1 = b_g1.reshape(1, DH)
    b2 = b_g2.reshape(1, D)

    scaled, seg, cnt = pl.pallas_call(
        _gating_body,
        grid=(nblk,),
        in_specs=[
            pl.BlockSpec((1, 1, BLK), lambda i: (i, 0, 0)),
            pl.BlockSpec((BLK, D), lambda i: (i, 0)),
            pl.BlockSpec((D, DH), lambda i: (0, 0)),
            pl.BlockSpec((1, DH), lambda i: (0, 0)),
            pl.BlockSpec((DH, D), lambda i: (0, 0)),
            pl.BlockSpec((1, D), lambda i: (0, 0)),
        ],
        out_specs=[
            pl.BlockSpec((BLK, D), lambda i: (i, 0)),
            pl.BlockSpec((G, D), lambda i: (0, 0)),
            pl.BlockSpec((G, 1), lambda i: (0, 0)),
        ],
        out_shape=[
            jax.ShapeDtypeStruct((npad, D), jnp.float32),
            jax.ShapeDtypeStruct((G, D), jnp.float32),
            jax.ShapeDtypeStruct((G, 1), jnp.float32),
        ],
        compiler_params=pltpu.CompilerParams(
            dimension_semantics=("arbitrary",)),
    )(idx3, x, w1t, b1, w2t, b2)

    gc = pl.pallas_call(
        _context_body,
        out_shape=jax.ShapeDtypeStruct((G, D), jnp.float32),
    )(seg, cnt, W_mean)

    out = pl.pallas_call(
        _output_body,
        grid=(nblk,),
        in_specs=[
            pl.BlockSpec((1, 1, BLK), lambda i: (i, 0, 0)),
            pl.BlockSpec((BLK, D), lambda i: (i, 0)),
            pl.BlockSpec((G, D), lambda i: (0, 0)),
        ],
        out_specs=pl.BlockSpec((G, D), lambda i: (0, 0)),
        out_shape=jax.ShapeDtypeStruct((G, D), jnp.float32),
        compiler_params=pltpu.CompilerParams(
            dimension_semantics=("arbitrary",)),
    )(idx3, scaled, gc)

    return out
